# baseline (speedup 1.0000x reference)
import functools

import jax
import jax.numpy as jnp
from jax import lax
from jax.experimental import pallas as pl
from jax.experimental.pallas import tpu as pltpu
from jax.experimental.pallas import tpu_sc as plsc

NUM_EMB = 100000
GENEPT_DIM = 1536
EMB_DIM = 128
EPS = 1e-5

ROW_BLOCK = 4000
CHUNK = 128


def _project_ln_block(t_ref, w_ref, b_ref, g_ref, be_ref, o_ref):
    y = jnp.dot(t_ref[...], w_ref[...],
                preferred_element_type=jnp.float32,
                precision=lax.Precision.DEFAULT)
    y = y + b_ref[...]
    mu = jnp.mean(y, axis=1, keepdims=True)
    d = y - mu
    var = jnp.mean(d * d, axis=1, keepdims=True)
    o_ref[...] = d * lax.rsqrt(var + EPS) * g_ref[...] + be_ref[...]


def _make_projected_table(table, W, b, gamma, beta):
    grid = NUM_EMB // ROW_BLOCK
    return pl.pallas_call(
        _project_ln_block,
        grid=(grid,),
        in_specs=[
            pl.BlockSpec((ROW_BLOCK, GENEPT_DIM), lambda i: (i, 0)),
            pl.BlockSpec((GENEPT_DIM, EMB_DIM), lambda i: (0, 0)),
            pl.BlockSpec((1, EMB_DIM), lambda i: (0, 0)),
            pl.BlockSpec((1, EMB_DIM), lambda i: (0, 0)),
            pl.BlockSpec((1, EMB_DIM), lambda i: (0, 0)),
        ],
        out_specs=pl.BlockSpec((ROW_BLOCK, EMB_DIM), lambda i: (i, 0)),
        out_shape=jax.ShapeDtypeStruct((NUM_EMB, EMB_DIM), jnp.float32),
    )(table, W, b.reshape(1, EMB_DIM), gamma.reshape(1, EMB_DIM),
      beta.reshape(1, EMB_DIM))


def _sc_gather(P, x2d, n_tokens):
    info = plsc.get_sparse_core_info()
    nw = info.num_cores * info.num_subcores
    n_chunks = n_tokens // CHUNK
    cpw = n_chunks // nw
    mesh = plsc.VectorSubcoreMesh(core_axis_name="c", subcore_axis_name="s")

    @functools.partial(
        pl.kernel, mesh=mesh,
        out_type=jax.ShapeDtypeStruct((n_tokens, EMB_DIM), jnp.float32),
        scratch_types=[
            pltpu.VMEM((cpw, CHUNK), jnp.int32),
            pltpu.VMEM((CHUNK, EMB_DIM), jnp.float32),
            pltpu.VMEM((CHUNK, EMB_DIM), jnp.float32),
            pltpu.SemaphoreType.DMA,
            pltpu.SemaphoreType.DMA,
        ],
    )
    def k(p_hbm, x_hbm, out_hbm, idx_v, buf0, buf1, sem0, sem1):
        wid = lax.axis_index("s") * info.num_cores + lax.axis_index("c")
        c0 = wid * cpw
        pltpu.sync_copy(x_hbm.at[pl.ds(c0, cpw)], idx_v)
        bufs = (buf0, buf1)
        sems = (sem0, sem1)

        def start(j, slot):
            pltpu.async_copy(p_hbm.at[idx_v.at[j]], bufs[slot], sems[slot])

        def finish(j, slot):
            pltpu.make_async_copy(p_hbm.at[pl.ds(0, CHUNK)], bufs[slot],
                                  sems[slot]).wait()
            pltpu.sync_copy(bufs[slot],
                            out_hbm.at[pl.ds((c0 + j) * CHUNK, CHUNK)])

        start(0, 0)
        start(1, 1)

        def body(g, carry):
            j = g * 2
            finish(j, 0)
            start(j + 2, 0)
            finish(j + 1, 1)
            start(j + 3, 1)
            return carry

        lax.fori_loop(0, cpw // 2 - 1, body, 0)
        finish(cpw - 2, 0)
        finish(cpw - 1, 1)

    return k(P, x2d)


def kernel(x, table, W, b, gamma, beta):
    P = _make_projected_table(table.astype(jnp.float32),
                              W.astype(jnp.float32),
                              b.astype(jnp.float32),
                              gamma.astype(jnp.float32),
                              beta.astype(jnp.float32))
    bsz, seq = x.shape
    n = bsz * seq
    x2d = x.astype(jnp.int32).reshape(n // CHUNK, CHUNK)
    out = _sc_gather(P, x2d, n)
    return out.reshape(bsz, seq, EMB_DIM)
